# baseline (speedup 1.0000x reference)
import jax
import jax.numpy as jnp
from jax import lax
from jax.experimental import pallas as pl
from jax.experimental.pallas import tpu as pltpu

_SPECIAL = 4
_LABEL_ID = 1
_ACT_V = 64
_TIME_V = 32
_C = _ACT_V + _TIME_V
_BLK = 512

_PREC = jax.lax.Precision.DEFAULT


def _proto_kernel(tok_ref, nxt_ref, *refs):
    h_refs = refs[:-6]
    e_ref, params_ref, out_ref, w_ref, sq_ref, l_ref = refs[-6:]
    b = pl.program_id(0)
    blk = _BLK

    s_ta = params_ref[0]
    s_tt = params_ref[1]
    s_pa = params_ref[2]
    s_pt = params_ref[3]
    alpha_a = params_ref[4]
    alpha_t = params_ref[5]
    tau_a = params_ref[6]
    tau_t = params_ref[7]

    col96 = lax.broadcasted_iota(jnp.int32, (1, _C), 1)
    is_act_col = col96 < _ACT_V
    tau_col = jnp.where(is_act_col, tau_a, tau_t)
    s_tied_col = jnp.where(is_act_col, s_ta, s_tt)
    s_proto_col = jnp.where(is_act_col, s_pa, s_pt)

    @pl.when(b == 0)
    def _init_once():
        rowi = lax.broadcasted_iota(jnp.int32, (blk, blk), 0)
        coli = lax.broadcasted_iota(jnp.int32, (blk, blk), 1)
        l_ref[...] = jnp.where(coli < rowi, 1.0, 0.0)
        w_ref[_C:2 * _C, :] = e_ref[...]

    e = e_ref[...]
    en = e / jnp.maximum(
        jnp.sqrt(jnp.sum(e * e, axis=1, keepdims=True)), 1e-12)
    row_idx = lax.broadcasted_iota(jnp.int32, (_C, 1), 0)
    alpha_row = jnp.where(row_idx < _ACT_V, alpha_a, alpha_t)
    w_ref[0:_C, :] = alpha_row * en
    col128 = lax.broadcasted_iota(jnp.int32, (1, 128), 1)
    alpha_col = jnp.where(col128 < _ACT_V, alpha_a, alpha_t)
    sq_ref[...] = jnp.where(col128 < _C, alpha_col * alpha_col, 0.0)

    ones_l = l_ref[...]

    for j, h_j in enumerate(h_refs):
        sl = slice(j * blk, (j + 1) * blk)
        h = h_j[0]
        hss = jnp.sum(h * h, axis=1, keepdims=True)
        inv = 1.0 / jnp.maximum(jnp.sqrt(hss), 1e-12)
        hn_ss = hss * inv * inv

        tok = tok_ref[0, sl, :]
        nxt = nxt_ref[0, sl, :]
        is_label = tok == _LABEL_ID
        sup_a = is_label & (nxt >= _SPECIAL) & (nxt < _SPECIAL + _ACT_V)
        sup_t = is_label & (nxt >= _SPECIAL + _ACT_V) & (nxt < _SPECIAL + _C)
        sup = sup_a | sup_t
        colc = lax.broadcasted_iota(jnp.int32, (blk, _C), 1)
        m = jnp.where(sup & (colc == nxt - _SPECIAL), 1.0, 0.0)
        mi = m * inv

        y = lax.dot_general(h, w_ref[...], (((1,), (1,)), ((), ())),
                            precision=_PREC)
        tied = y[:, _C:2 * _C]
        gram = lax.dot_general(h, h, (((1,), (1,)), ((), ())),
                               precision=_PREC)
        gram_l = gram * ones_l
        num = inv * (y[:, :_C] +
                     lax.dot_general(gram_l, mi, (((1,), (0,)), ((), ())),
                                     precision=_PREC))

        inc = m * (2.0 * num + hn_ss)
        sup_af = jnp.where(sup_a, 1.0, 0.0)
        sup_tf = jnp.where(sup_t, 1.0, 0.0)
        x = jnp.concatenate(
            [inc, sup_af, sup_tf,
             jnp.zeros((blk, 128 - _C - 2), jnp.float32)],
            axis=1)
        cum = lax.dot_general(ones_l, x, (((1,), (0,)), ((), ())),
                              precision=_PREC)
        base = sq_ref[...]
        sqnorm = base[:, :_C] + cum[:, :_C]
        cnts = base[:, _C:_C + 2] + cum[:, _C:_C + 2]

        denom = jnp.maximum(jnp.sqrt(jnp.maximum(sqnorm, 0.0)), 1e-12)
        gate_cnt = jnp.where(is_act_col, cnts[:, 0:1], cnts[:, 1:2])
        gate = is_label & (gate_cnt > 0.0)
        proto = jnp.where(gate, num / denom * tau_col, 0.0)

        out_ref[0, sl, :] = s_tied_col * tied + s_proto_col * proto

        w_ref[0:_C, :] = w_ref[0:_C, :] + lax.dot_general(
            mi, h, (((0,), (0,)), ((), ())), precision=_PREC)
        sq_ref[...] = base + jnp.sum(x, axis=0, keepdims=True)


def kernel(h, E, tokens, tied_scale_act, tied_scale_time, proto_scale_act,
           proto_scale_time, proto_prior_act, proto_prior_time,
           proto_temp_act, proto_temp_time):
    b, t, d = h.shape
    n_chunks = t // _BLK

    params = jnp.stack([
        jax.nn.softplus(tied_scale_act),
        jax.nn.softplus(tied_scale_time),
        jax.nn.softplus(proto_scale_act),
        jax.nn.softplus(proto_scale_time),
        jax.nn.softplus(proto_prior_act),
        jax.nn.softplus(proto_prior_time),
        jax.nn.softplus(proto_temp_act),
        jax.nn.softplus(proto_temp_time),
    ]).astype(jnp.float32)

    tokens = tokens.astype(jnp.int32)
    nxt = jnp.roll(tokens, -1, axis=1)
    tok3 = tokens.reshape(b, t, 1)
    nxt3 = nxt.reshape(b, t, 1)
    e_sub = E[_SPECIAL:_SPECIAL + _C].astype(jnp.float32)

    def _chunk_spec(k):
        return pl.BlockSpec((1, _BLK, d), lambda i, _k=k: (i, _k, 0))

    out = pl.pallas_call(
        _proto_kernel,
        grid=(b,),
        in_specs=[
            pl.BlockSpec((1, t, 1), lambda i: (i, 0, 0)),
            pl.BlockSpec((1, t, 1), lambda i: (i, 0, 0)),
            *[_chunk_spec(k) for k in range(n_chunks)],
            pl.BlockSpec((_C, d), lambda i: (0, 0)),
            pl.BlockSpec(memory_space=pltpu.SMEM),
        ],
        out_specs=pl.BlockSpec((1, t, _C), lambda i: (i, 0, 0)),
        out_shape=jax.ShapeDtypeStruct((b, t, _C), jnp.float32),
        scratch_shapes=[
            pltpu.VMEM((2 * _C, d), jnp.float32),
            pltpu.VMEM((1, 128), jnp.float32),
            pltpu.VMEM((_BLK, _BLK), jnp.float32),
        ],
    )(tok3, nxt3, *([h.astype(jnp.float32)] * n_chunks), e_sub, params)
    return out


# speedup vs baseline: 1.0850x; 1.0850x over previous
import jax
import jax.numpy as jnp
from jax import lax
from jax.experimental import pallas as pl
from jax.experimental.pallas import tpu as pltpu

_SPECIAL = 4
_LABEL_ID = 1
_ACT_V = 64
_TIME_V = 32
_C = _ACT_V + _TIME_V
_BLK = 512

_PREC = jax.lax.Precision.DEFAULT


def _proto_kernel(tok_ref, h_ref, e_ref, params_ref,
                  out_ref, w_ref, sq_ref, l_ref):
    b = pl.program_id(0)
    blk = _BLK
    t_len = h_ref.shape[1]
    n_chunks = t_len // blk

    s_ta = params_ref[0]
    s_tt = params_ref[1]
    s_pa = params_ref[2]
    s_pt = params_ref[3]
    alpha_a = params_ref[4]
    alpha_t = params_ref[5]
    tau_a = params_ref[6]
    tau_t = params_ref[7]

    col96 = lax.broadcasted_iota(jnp.int32, (1, _C), 1)
    is_act_col = col96 < _ACT_V
    tau_col = jnp.where(is_act_col, tau_a, tau_t)
    s_tied_col = jnp.where(is_act_col, s_ta, s_tt)
    s_proto_col = jnp.where(is_act_col, s_pa, s_pt)

    @pl.when(b == 0)
    def _init_once():
        rowi = lax.broadcasted_iota(jnp.int32, (blk, blk), 0)
        coli = lax.broadcasted_iota(jnp.int32, (blk, blk), 1)
        l_ref[...] = jnp.where(coli < rowi, 1.0, 0.0)
        w_ref[_C:2 * _C, :] = e_ref[...]

    e = e_ref[...]
    en = e / jnp.maximum(
        jnp.sqrt(jnp.sum(e * e, axis=1, keepdims=True)), 1e-12)
    row_idx = lax.broadcasted_iota(jnp.int32, (_C, 1), 0)
    alpha_row = jnp.where(row_idx < _ACT_V, alpha_a, alpha_t)
    w_ref[0:_C, :] = alpha_row * en
    col128 = lax.broadcasted_iota(jnp.int32, (1, 128), 1)
    alpha_col = jnp.where(col128 < _ACT_V, alpha_a, alpha_t)
    sq_ref[...] = jnp.where(col128 < _C, alpha_col * alpha_col, 0.0)

    ones_l = l_ref[...]

    for j in range(n_chunks):
        sl = slice(j * blk, (j + 1) * blk)
        h = h_ref[0, sl, :]
        hss = jnp.sum(h * h, axis=1, keepdims=True)
        inv = 1.0 / jnp.maximum(jnp.sqrt(hss), 1e-12)
        hn_ss = hss * inv * inv

        tok = tok_ref[0, sl, :]
        wrap = tok_ref[0, (j + 1) * blk:(j + 1) * blk + 1, :] \
            if j + 1 < n_chunks else tok_ref[0, 0:1, :]
        nxt = jnp.concatenate(
            [tok_ref[0, j * blk + 1:(j + 1) * blk, :], wrap], axis=0)
        is_label = tok == _LABEL_ID
        sup_a = is_label & (nxt >= _SPECIAL) & (nxt < _SPECIAL + _ACT_V)
        sup_t = is_label & (nxt >= _SPECIAL + _ACT_V) & (nxt < _SPECIAL + _C)
        sup = sup_a | sup_t
        colc = lax.broadcasted_iota(jnp.int32, (blk, _C), 1)
        m = jnp.where(sup & (colc == nxt - _SPECIAL), 1.0, 0.0)
        mi = m * inv

        y = lax.dot_general(h, w_ref[...], (((1,), (1,)), ((), ())),
                            precision=_PREC)
        tied = y[:, _C:2 * _C]
        gram = lax.dot_general(h, h, (((1,), (1,)), ((), ())),
                               precision=_PREC)
        gram_l = gram * ones_l
        num = inv * (y[:, :_C] +
                     lax.dot_general(gram_l, mi, (((1,), (0,)), ((), ())),
                                     precision=_PREC))

        inc = m * (2.0 * num + hn_ss)
        sup_af = jnp.where(sup_a, 1.0, 0.0)
        sup_tf = jnp.where(sup_t, 1.0, 0.0)
        x = jnp.concatenate(
            [inc, sup_af, sup_tf,
             jnp.zeros((blk, 128 - _C - 2), jnp.float32)],
            axis=1)
        cum = lax.dot_general(ones_l, x, (((1,), (0,)), ((), ())),
                              precision=_PREC)
        base = sq_ref[...]
        sqnorm = base[:, :_C] + cum[:, :_C]
        cnts = base[:, _C:_C + 2] + cum[:, _C:_C + 2]

        denom = jnp.maximum(jnp.sqrt(jnp.maximum(sqnorm, 0.0)), 1e-12)
        gate_cnt = jnp.where(is_act_col, cnts[:, 0:1], cnts[:, 1:2])
        gate = is_label & (gate_cnt > 0.0)
        proto = jnp.where(gate, num / denom * tau_col, 0.0)

        out_ref[0, sl, :] = s_tied_col * tied + s_proto_col * proto

        w_ref[0:_C, :] = w_ref[0:_C, :] + lax.dot_general(
            mi, h, (((0,), (0,)), ((), ())), precision=_PREC)
        sq_ref[...] = base + jnp.sum(x, axis=0, keepdims=True)


def kernel(h, E, tokens, tied_scale_act, tied_scale_time, proto_scale_act,
           proto_scale_time, proto_prior_act, proto_prior_time,
           proto_temp_act, proto_temp_time):
    b, t, d = h.shape

    params = jnp.stack([
        jax.nn.softplus(tied_scale_act),
        jax.nn.softplus(tied_scale_time),
        jax.nn.softplus(proto_scale_act),
        jax.nn.softplus(proto_scale_time),
        jax.nn.softplus(proto_prior_act),
        jax.nn.softplus(proto_prior_time),
        jax.nn.softplus(proto_temp_act),
        jax.nn.softplus(proto_temp_time),
    ]).astype(jnp.float32)

    tok3 = tokens.reshape(b, t, 1)
    e_sub = E[_SPECIAL:_SPECIAL + _C]

    out = pl.pallas_call(
        _proto_kernel,
        grid=(b,),
        in_specs=[
            pl.BlockSpec((1, t, 1), lambda i: (i, 0, 0)),
            pl.BlockSpec((1, t, d), lambda i: (i, 0, 0)),
            pl.BlockSpec((_C, d), lambda i: (0, 0)),
            pl.BlockSpec(memory_space=pltpu.SMEM),
        ],
        out_specs=pl.BlockSpec((1, t, _C), lambda i: (i, 0, 0)),
        out_shape=jax.ShapeDtypeStruct((b, t, _C), jnp.float32),
        scratch_shapes=[
            pltpu.VMEM((2 * _C, d), jnp.float32),
            pltpu.VMEM((1, 128), jnp.float32),
            pltpu.VMEM((_BLK, _BLK), jnp.float32),
        ],
    )(tok3, h, e_sub, params)
    return out


# speedup vs baseline: 1.2060x; 1.1115x over previous
import jax
import jax.numpy as jnp
from jax import lax
from jax.experimental import pallas as pl
from jax.experimental.pallas import tpu as pltpu

_SPECIAL = 4
_LABEL_ID = 1
_ACT_V = 64
_TIME_V = 32
_C = _ACT_V + _TIME_V
_BLK = 512

_PREC = jax.lax.Precision.DEFAULT


def _proto_kernel(tok_ref, h_ref, e_ref, params_ref,
                  out_ref, w_ref, sq_ref, l_ref):
    b = pl.program_id(0)
    blk = _BLK
    t_len = h_ref.shape[1]
    n_chunks = t_len // blk

    s_ta = params_ref[0]
    s_tt = params_ref[1]
    s_pa = params_ref[2]
    s_pt = params_ref[3]
    alpha_a = params_ref[4]
    alpha_t = params_ref[5]
    tau_a = params_ref[6]
    tau_t = params_ref[7]

    col96 = lax.broadcasted_iota(jnp.int32, (1, _C), 1)
    is_act_col = col96 < _ACT_V
    tau_col = jnp.where(is_act_col, tau_a, tau_t)
    s_tied_col = jnp.where(is_act_col, s_ta, s_tt)
    s_proto_col = jnp.where(is_act_col, s_pa, s_pt)

    @pl.when(b == 0)
    def _init_once():
        rowi = lax.broadcasted_iota(jnp.int32, (blk, blk), 0)
        coli = lax.broadcasted_iota(jnp.int32, (blk, blk), 1)
        l_ref[...] = jnp.where(coli < rowi, 1.0, 0.0)
        w_ref[_C:2 * _C, :] = e_ref[...]

    e = e_ref[...]
    en = e / jnp.maximum(
        jnp.sqrt(jnp.sum(e * e, axis=1, keepdims=True)), 1e-12)
    row_idx = lax.broadcasted_iota(jnp.int32, (_C, 1), 0)
    alpha_row = jnp.where(row_idx < _ACT_V, alpha_a, alpha_t)
    w_ref[0:_C, :] = alpha_row * en
    col128 = lax.broadcasted_iota(jnp.int32, (1, 128), 1)
    alpha_col = jnp.where(col128 < _ACT_V, alpha_a, alpha_t)
    sq_ref[...] = jnp.where(col128 < _C, alpha_col * alpha_col, 0.0)

    ones_l = l_ref[...]

    for j in range(n_chunks):
        sl = slice(j * blk, (j + 1) * blk)
        h = h_ref[0, sl, :]
        hss = jnp.sum(h * h, axis=1, keepdims=True)
        inv = 1.0 / jnp.maximum(jnp.sqrt(hss), 1e-12)
        hn_ss = hss * inv * inv

        tok = tok_ref[0, sl, :]
        wrap = tok_ref[0, (j + 1) * blk:(j + 1) * blk + 1, :] \
            if j + 1 < n_chunks else tok_ref[0, 0:1, :]
        nxt = jnp.concatenate(
            [tok_ref[0, j * blk + 1:(j + 1) * blk, :], wrap], axis=0)
        is_label = tok == _LABEL_ID
        sup_a = is_label & (nxt >= _SPECIAL) & (nxt < _SPECIAL + _ACT_V)
        sup_t = is_label & (nxt >= _SPECIAL + _ACT_V) & (nxt < _SPECIAL + _C)
        sup = sup_a | sup_t
        colc = lax.broadcasted_iota(jnp.int32, (blk, _C), 1)
        m = jnp.where(sup & (colc == nxt - _SPECIAL), 1.0, 0.0)
        mi = m * inv

        y = lax.dot_general(h, w_ref[...], (((1,), (1,)), ((), ())),
                            precision=_PREC)
        tied = y[:, _C:2 * _C]
        gram = lax.dot_general(h, h, (((1,), (1,)), ((), ())),
                               precision=_PREC)
        gram_l = gram * ones_l
        num = inv * (y[:, :_C] +
                     lax.dot_general(gram_l, mi, (((1,), (0,)), ((), ())),
                                     precision=_PREC))

        inc = m * (2.0 * num + hn_ss)
        sup_af = jnp.where(sup_a, 1.0, 0.0)
        sup_tf = jnp.where(sup_t, 1.0, 0.0)
        x = jnp.concatenate(
            [inc, sup_af, sup_tf,
             jnp.zeros((blk, 128 - _C - 2), jnp.float32)],
            axis=1)
        cum = lax.dot_general(ones_l, x, (((1,), (0,)), ((), ())),
                              precision=_PREC)
        base = sq_ref[...]
        sqnorm = base[:, :_C] + cum[:, :_C]
        cnts = base[:, _C:_C + 2] + cum[:, _C:_C + 2]

        denom = jnp.maximum(jnp.sqrt(jnp.maximum(sqnorm, 0.0)), 1e-12)
        gate_cnt = jnp.where(is_act_col, cnts[:, 0:1], cnts[:, 1:2])
        gate = is_label & (gate_cnt > 0.0)
        proto = jnp.where(gate, num / denom * tau_col, 0.0)

        res = s_tied_col * tied + s_proto_col * proto
        out_ref[0, :, sl] = jnp.transpose(res)

        w_ref[0:_C, :] = w_ref[0:_C, :] + lax.dot_general(
            mi, h, (((0,), (0,)), ((), ())), precision=_PREC)
        sq_ref[...] = base + jnp.sum(x, axis=0, keepdims=True)


def kernel(h, E, tokens, tied_scale_act, tied_scale_time, proto_scale_act,
           proto_scale_time, proto_prior_act, proto_prior_time,
           proto_temp_act, proto_temp_time):
    b, t, d = h.shape

    params = jnp.stack([
        jax.nn.softplus(tied_scale_act),
        jax.nn.softplus(tied_scale_time),
        jax.nn.softplus(proto_scale_act),
        jax.nn.softplus(proto_scale_time),
        jax.nn.softplus(proto_prior_act),
        jax.nn.softplus(proto_prior_time),
        jax.nn.softplus(proto_temp_act),
        jax.nn.softplus(proto_temp_time),
    ]).astype(jnp.float32)

    tok3 = tokens.reshape(b, t, 1)
    e_sub = E[_SPECIAL:_SPECIAL + _C]

    out = pl.pallas_call(
        _proto_kernel,
        grid=(b,),
        in_specs=[
            pl.BlockSpec((1, t, 1), lambda i: (i, 0, 0)),
            pl.BlockSpec((1, t, d), lambda i: (i, 0, 0)),
            pl.BlockSpec((_C, d), lambda i: (0, 0)),
            pl.BlockSpec(memory_space=pltpu.SMEM),
        ],
        out_specs=pl.BlockSpec((1, _C, t), lambda i: (i, 0, 0)),
        out_shape=jax.ShapeDtypeStruct((b, _C, t), jnp.float32),
        scratch_shapes=[
            pltpu.VMEM((2 * _C, d), jnp.float32),
            pltpu.VMEM((1, 128), jnp.float32),
            pltpu.VMEM((_BLK, _BLK), jnp.float32),
        ],
    )(tok3, h, e_sub, params)
    return jnp.transpose(out, (0, 2, 1))


# speedup vs baseline: 1.3424x; 1.1131x over previous
import jax
import jax.numpy as jnp
from jax import lax
from jax.experimental import pallas as pl
from jax.experimental.pallas import tpu as pltpu

_SPECIAL = 4
_LABEL_ID = 1
_ACT_V = 64
_TIME_V = 32
_C = _ACT_V + _TIME_V
_BLK = 512

_PREC = jax.lax.Precision.DEFAULT


def _proto_kernel(tok_ref, h_ref, e_ref, params_ref,
                  out_ref, w_ref, sq_ref, l_ref):
    b = pl.program_id(0)
    blk = _BLK
    t_len = h_ref.shape[1]
    n_chunks = t_len // blk

    s_ta = params_ref[0]
    s_tt = params_ref[1]
    s_pa = params_ref[2]
    s_pt = params_ref[3]
    alpha_a = params_ref[4]
    alpha_t = params_ref[5]
    tau_a = params_ref[6]
    tau_t = params_ref[7]

    col96 = lax.broadcasted_iota(jnp.int32, (1, _C), 1)
    is_act_col = col96 < _ACT_V
    tau_col = jnp.where(is_act_col, tau_a, tau_t)
    s_tied_col = jnp.where(is_act_col, s_ta, s_tt)
    s_proto_col = jnp.where(is_act_col, s_pa, s_pt)

    @pl.when(b == 0)
    def _init_once():
        rowi = lax.broadcasted_iota(jnp.int32, (blk, blk), 0)
        coli = lax.broadcasted_iota(jnp.int32, (blk, blk), 1)
        l_ref[...] = jnp.where(coli < rowi, 1.0, 0.0)
        w_ref[_C:2 * _C, :] = e_ref[_SPECIAL:_SPECIAL + _C, :]

    e = e_ref[_SPECIAL:_SPECIAL + _C, :]
    en = e / jnp.maximum(
        jnp.sqrt(jnp.sum(e * e, axis=1, keepdims=True)), 1e-12)
    row_idx = lax.broadcasted_iota(jnp.int32, (_C, 1), 0)
    alpha_row = jnp.where(row_idx < _ACT_V, alpha_a, alpha_t)
    w_ref[0:_C, :] = alpha_row * en
    col128 = lax.broadcasted_iota(jnp.int32, (1, 128), 1)
    alpha_col = jnp.where(col128 < _ACT_V, alpha_a, alpha_t)
    sq_ref[...] = jnp.where(col128 < _C, alpha_col * alpha_col, 0.0)

    ones_l = l_ref[...]
    tok_seq = tok_ref[pl.ds(b, 1), :]

    for j in range(n_chunks):
        sl = slice(j * blk, (j + 1) * blk)
        h = h_ref[0, sl, :]
        hss = jnp.sum(h * h, axis=1, keepdims=True)
        inv = 1.0 / jnp.maximum(jnp.sqrt(hss), 1e-12)
        hn_ss = hss * inv * inv

        tok_row = tok_seq[:, sl]
        wrap = tok_seq[:, (j + 1) * blk:(j + 1) * blk + 1] \
            if j + 1 < n_chunks else tok_seq[:, 0:1]
        nxt_row = jnp.concatenate(
            [tok_seq[:, j * blk + 1:(j + 1) * blk], wrap], axis=1)
        tok = jnp.transpose(tok_row)
        nxt = jnp.transpose(nxt_row)
        is_label = tok == _LABEL_ID
        sup_a = is_label & (nxt >= _SPECIAL) & (nxt < _SPECIAL + _ACT_V)
        sup_t = is_label & (nxt >= _SPECIAL + _ACT_V) & (nxt < _SPECIAL + _C)
        sup = sup_a | sup_t
        colc = lax.broadcasted_iota(jnp.int32, (blk, _C), 1)
        m = jnp.where(sup & (colc == nxt - _SPECIAL), 1.0, 0.0)
        mi = m * inv

        y = lax.dot_general(h, w_ref[...], (((1,), (1,)), ((), ())),
                            precision=_PREC)
        tied = y[:, _C:2 * _C]
        gram = lax.dot_general(h, h, (((1,), (1,)), ((), ())),
                               precision=_PREC)
        gram_l = gram * ones_l
        num = inv * (y[:, :_C] +
                     lax.dot_general(gram_l, mi, (((1,), (0,)), ((), ())),
                                     precision=_PREC))

        inc = m * (2.0 * num + hn_ss)
        sup_af = jnp.where(sup_a, 1.0, 0.0)
        sup_tf = jnp.where(sup_t, 1.0, 0.0)
        x = jnp.concatenate(
            [inc, sup_af, sup_tf,
             jnp.zeros((blk, 128 - _C - 2), jnp.float32)],
            axis=1)
        cum = lax.dot_general(ones_l, x, (((1,), (0,)), ((), ())),
                              precision=_PREC)
        base = sq_ref[...]
        sqnorm = base[:, :_C] + cum[:, :_C]
        cnts = base[:, _C:_C + 2] + cum[:, _C:_C + 2]

        denom = jnp.maximum(jnp.sqrt(jnp.maximum(sqnorm, 0.0)), 1e-12)
        gate_cnt = jnp.where(is_act_col, cnts[:, 0:1], cnts[:, 1:2])
        gate = is_label & (gate_cnt > 0.0)
        proto = jnp.where(gate, num / denom * tau_col, 0.0)

        res = s_tied_col * tied + s_proto_col * proto
        out_ref[0, :, sl] = jnp.transpose(res)

        w_ref[0:_C, :] = w_ref[0:_C, :] + lax.dot_general(
            mi, h, (((0,), (0,)), ((), ())), precision=_PREC)
        sq_ref[...] = base + jnp.sum(x, axis=0, keepdims=True)


def kernel(h, E, tokens, tied_scale_act, tied_scale_time, proto_scale_act,
           proto_scale_time, proto_prior_act, proto_prior_time,
           proto_temp_act, proto_temp_time):
    b, t, d = h.shape

    params = jnp.stack([
        jax.nn.softplus(tied_scale_act),
        jax.nn.softplus(tied_scale_time),
        jax.nn.softplus(proto_scale_act),
        jax.nn.softplus(proto_scale_time),
        jax.nn.softplus(proto_prior_act),
        jax.nn.softplus(proto_prior_time),
        jax.nn.softplus(proto_temp_act),
        jax.nn.softplus(proto_temp_time),
    ]).astype(jnp.float32)

    ev = E.shape[0]

    out = pl.pallas_call(
        _proto_kernel,
        grid=(b,),
        in_specs=[
            pl.BlockSpec((b, t), lambda i: (0, 0)),
            pl.BlockSpec((1, t, d), lambda i: (i, 0, 0)),
            pl.BlockSpec((ev, d), lambda i: (0, 0)),
            pl.BlockSpec(memory_space=pltpu.SMEM),
        ],
        out_specs=pl.BlockSpec((1, _C, t), lambda i: (i, 0, 0)),
        out_shape=jax.ShapeDtypeStruct((b, _C, t), jnp.float32),
        scratch_shapes=[
            pltpu.VMEM((2 * _C, d), jnp.float32),
            pltpu.VMEM((1, 128), jnp.float32),
            pltpu.VMEM((_BLK, _BLK), jnp.float32),
        ],
    )(tokens, h, E, params)
    return jnp.transpose(out, (0, 2, 1))
